# Initial kernel scaffold; baseline (speedup 1.0000x reference)
import jax
import jax.numpy as jnp
from jax.experimental import pallas as pl


def kernel(x, edge_index, edge_attr, eps, W1, b1, gamma, beta, W2, b2):
    raise NotImplementedError("write your pallas kernel here")


# speedup vs baseline: 3.4052x; 3.4052x over previous
import functools

import jax
import jax.numpy as jnp
from jax import lax
from jax.experimental import pallas as pl
from jax.experimental.pallas import tpu as pltpu
from jax.experimental.pallas import tpu_sc as plsc

N_NODES = 10000
N_EDGES = 320000
DIM = 128

NC = 2
NS = 16
NW = NC * NS
E_PER_W = N_EDGES // NW
CHUNK = 80
N_CHUNKS = E_PER_W // CHUNK
STRIPE = 80
N_STRIPES = N_NODES // STRIPE
STRIPES_PER_TILE = -(-N_STRIPES // NS)
LANES = 16


def _sc_edge_body(x_hbm, src_hbm, dst_hbm, ea_hbm, out_hbm,
                  src_v, dst_v, rows_v, ea_v, zb_v, acc_sh, sem):
    cid = lax.axis_index("c")
    sid = lax.axis_index("s")
    wid = sid * NC + cid

    def zero_row(r, _):
        for j in range(DIM // LANES):
            zb_v[r, pl.ds(j * LANES, LANES)] = jnp.zeros((LANES,), jnp.float32)
        return 0
    lax.fori_loop(0, STRIPE, zero_row, 0)
    for k in range(STRIPES_PER_TILE):
        s = sid + k * NS
        @pl.when(s < N_STRIPES)
        def _():
            pltpu.sync_copy(zb_v, acc_sh.at[pl.ds(s * STRIPE, STRIPE)])
    plsc.subcore_barrier()

    base0 = wid * E_PER_W

    def chunk_body(i, _):
        base = base0 + i * CHUNK
        pltpu.sync_copy(src_hbm.at[pl.ds(base, CHUNK)], src_v)
        pltpu.sync_copy(dst_hbm.at[pl.ds(base, CHUNK)], dst_v)
        pltpu.sync_copy(ea_hbm.at[pl.ds(base, CHUNK)], ea_v)
        pltpu.async_copy(x_hbm.at[src_v], rows_v, sem).wait()

        def row_body(r, _):
            for j in range(DIM // LANES):
                sl = pl.ds(j * LANES, LANES)
                rows_v[r, sl] = jnp.maximum(rows_v[r, sl] + ea_v[r, sl], 0.0)
            return 0
        lax.fori_loop(0, CHUNK, row_body, 0)

        pltpu.sync_copy(rows_v, acc_sh.at[dst_v], add=True)
        return 0

    lax.fori_loop(0, N_CHUNKS, chunk_body, 0)
    plsc.subcore_barrier()

    for k in range(STRIPES_PER_TILE):
        s = sid + k * NS
        @pl.when(s < N_STRIPES)
        def _():
            pltpu.sync_copy(acc_sh.at[pl.ds(s * STRIPE, STRIPE)],
                            out_hbm.at[cid, pl.ds(s * STRIPE, STRIPE)])


_sc_edge = functools.partial(
    pl.kernel,
    out_type=jax.ShapeDtypeStruct((NC, N_NODES, DIM), jnp.float32),
    mesh=plsc.VectorSubcoreMesh(core_axis_name="c", subcore_axis_name="s",
                                num_cores=NC, num_subcores=NS),
    scratch_types=[
        pltpu.VMEM((CHUNK,), jnp.int32),
        pltpu.VMEM((CHUNK,), jnp.int32),
        pltpu.VMEM((CHUNK, DIM), jnp.float32),
        pltpu.VMEM((CHUNK, DIM), jnp.float32),
        pltpu.VMEM((STRIPE, DIM), jnp.float32),
        pltpu.VMEM_SHARED((N_NODES, DIM), jnp.float32),
        pltpu.SemaphoreType.DMA,
    ],
)(_sc_edge_body)


def _tc_mlp_body(eps_ref, x_ref, p_ref, w1_ref, b1_ref, g_ref, be_ref,
                 w2_ref, b2_ref, o_ref):
    h = x_ref[...] * (1.0 + eps_ref[0]) + p_ref[0] + p_ref[1]
    h1 = lax.dot_general(h, w1_ref[...], (((1,), (1,)), ((), ())),
                         preferred_element_type=jnp.float32) + b1_ref[...]
    mean = jnp.mean(h1, axis=0, keepdims=True)
    var = jnp.mean(jnp.square(h1 - mean), axis=0, keepdims=True)
    h2 = (h1 - mean) * lax.rsqrt(var + 1e-5) * g_ref[...] + be_ref[...]
    h2 = jnp.maximum(h2, 0.0)
    o_ref[...] = lax.dot_general(h2, w2_ref[...], (((1,), (1,)), ((), ())),
                                 preferred_element_type=jnp.float32) + b2_ref[...]


def _tc_mlp(eps, x, partials, w1, b1, gamma, beta, w2, b2):
    return pl.pallas_call(
        _tc_mlp_body,
        out_shape=jax.ShapeDtypeStruct((N_NODES, DIM), jnp.float32),
        in_specs=[
            pl.BlockSpec(memory_space=pltpu.SMEM),
            pl.BlockSpec(memory_space=pltpu.VMEM),
            pl.BlockSpec(memory_space=pltpu.VMEM),
            pl.BlockSpec(memory_space=pltpu.VMEM),
            pl.BlockSpec(memory_space=pltpu.VMEM),
            pl.BlockSpec(memory_space=pltpu.VMEM),
            pl.BlockSpec(memory_space=pltpu.VMEM),
            pl.BlockSpec(memory_space=pltpu.VMEM),
            pl.BlockSpec(memory_space=pltpu.VMEM),
        ],
        out_specs=pl.BlockSpec(memory_space=pltpu.VMEM),
    )(eps, x, partials, w1, b1, gamma, beta, w2, b2)


def kernel(x, edge_index, edge_attr, eps, W1, b1, gamma, beta, W2, b2):
    dst = edge_index[0]
    src = edge_index[1]
    partials = _sc_edge(x, src, dst, edge_attr)
    return _tc_mlp(eps, x, partials, W1,
                   b1.reshape(1, DIM), gamma.reshape(1, DIM),
                   beta.reshape(1, DIM), W2, b2.reshape(1, DIM))


# speedup vs baseline: 7.3016x; 2.1443x over previous
import functools

import jax
import jax.numpy as jnp
from jax import lax
from jax.experimental import pallas as pl
from jax.experimental.pallas import tpu as pltpu
from jax.experimental.pallas import tpu_sc as plsc

N_NODES = 10000
N_EDGES = 320000
DIM = 128

NC = 2
NS = 16
NW = NC * NS
E_PER_W = N_EDGES // NW
CHUNK = 80
N_CHUNKS = E_PER_W // CHUNK
STRIPE = 80
N_STRIPES = N_NODES // STRIPE
STRIPES_PER_TILE = -(-N_STRIPES // NS)
LANES = 16


NBUF = 2


def _sc_edge_body(x_hbm, src_hbm, dst_hbm, ea_hbm, out_hbm,
                  src_bufs, dst_bufs, rows_bufs, ea_bufs, acc_sh,
                  isems, esems, gsems, ssems):
    cid = lax.axis_index("c")
    sid = lax.axis_index("s")
    wid = sid * NC + cid

    def zero_row(r, _):
        for j in range(DIM // LANES):
            rows_bufs[0][r, pl.ds(j * LANES, LANES)] = (
                jnp.zeros((LANES,), jnp.float32))
        return 0
    lax.fori_loop(0, STRIPE, zero_row, 0)
    for k in range(STRIPES_PER_TILE):
        s = sid + k * NS
        @pl.when(s < N_STRIPES)
        def _():
            pltpu.sync_copy(rows_bufs[0], acc_sh.at[pl.ds(s * STRIPE, STRIPE)])
    plsc.subcore_barrier()

    base0 = wid * E_PER_W

    def start_loads(i, b):
        base = base0 + i * CHUNK
        pltpu.async_copy(src_hbm.at[pl.ds(base, CHUNK)], src_bufs[b], isems[b])
        pltpu.async_copy(dst_hbm.at[pl.ds(base, CHUNK)], dst_bufs[b], isems[b])
        pltpu.async_copy(ea_hbm.at[pl.ds(base, CHUNK)], ea_bufs[b], esems[b])

    def wait_idx_and_gather(b):
        pltpu.make_async_copy(
            src_hbm.at[pl.ds(0, CHUNK)], src_bufs[b], isems[b]).wait()
        pltpu.make_async_copy(
            dst_hbm.at[pl.ds(0, CHUNK)], dst_bufs[b], isems[b]).wait()
        pltpu.async_copy(x_hbm.at[src_bufs[b]], rows_bufs[b], gsems[b])

    def compute_and_scatter(b):
        pltpu.make_async_copy(
            x_hbm.at[src_bufs[b]], rows_bufs[b], gsems[b]).wait()
        pltpu.make_async_copy(
            ea_hbm.at[pl.ds(0, CHUNK)], ea_bufs[b], esems[b]).wait()

        def row_body(r, _):
            for j in range(DIM // LANES):
                sl = pl.ds(j * LANES, LANES)
                rows_bufs[b][r, sl] = jnp.maximum(
                    rows_bufs[b][r, sl] + ea_bufs[b][r, sl], 0.0)
            return 0
        lax.fori_loop(0, CHUNK, row_body, 0)
        pltpu.async_copy(rows_bufs[b], acc_sh.at[dst_bufs[b]], ssems[b],
                         add=True)

    start_loads(0, 0)
    wait_idx_and_gather(0)

    def ring_body(g, _):
        i0 = g * NBUF
        for j in range(NBUF):
            i = i0 + j
            bn = (j + 1) % NBUF

            @pl.when(i >= NBUF - 1)
            def _():
                pltpu.make_async_copy(
                    rows_bufs[bn], acc_sh.at[dst_bufs[bn]], ssems[bn]).wait()

            @pl.when(i + 1 < N_CHUNKS)
            def _():
                start_loads(i + 1, bn)
                wait_idx_and_gather(bn)

            compute_and_scatter(j)
        return 0

    lax.fori_loop(0, (N_CHUNKS - 1) // NBUF, ring_body, 0)
    compute_and_scatter(0)
    for j in range(NBUF):
        pltpu.make_async_copy(
            rows_bufs[j], acc_sh.at[dst_bufs[j]], ssems[j]).wait()
    plsc.subcore_barrier()

    for k in range(STRIPES_PER_TILE):
        s = sid + k * NS
        @pl.when(s < N_STRIPES)
        def _():
            pltpu.sync_copy(acc_sh.at[pl.ds(s * STRIPE, STRIPE)],
                            out_hbm.at[cid, pl.ds(s * STRIPE, STRIPE)])


_sc_edge = functools.partial(
    pl.kernel,
    out_type=jax.ShapeDtypeStruct((NC, N_NODES, DIM), jnp.float32),
    mesh=plsc.VectorSubcoreMesh(core_axis_name="c", subcore_axis_name="s",
                                num_cores=NC, num_subcores=NS),
    scratch_types=[
        tuple(pltpu.VMEM((CHUNK,), jnp.int32) for _ in range(NBUF)),
        tuple(pltpu.VMEM((CHUNK,), jnp.int32) for _ in range(NBUF)),
        tuple(pltpu.VMEM((CHUNK, DIM), jnp.float32) for _ in range(NBUF)),
        tuple(pltpu.VMEM((CHUNK, DIM), jnp.float32) for _ in range(NBUF)),
        pltpu.VMEM_SHARED((N_NODES, DIM), jnp.float32),
        tuple(pltpu.SemaphoreType.DMA for _ in range(NBUF)),
        tuple(pltpu.SemaphoreType.DMA for _ in range(NBUF)),
        tuple(pltpu.SemaphoreType.DMA for _ in range(NBUF)),
        tuple(pltpu.SemaphoreType.DMA for _ in range(NBUF)),
    ],
)(_sc_edge_body)


def _tc_mlp_body(eps_ref, x_ref, p_ref, w1_ref, b1_ref, g_ref, be_ref,
                 w2_ref, b2_ref, o_ref):
    h = x_ref[...] * (1.0 + eps_ref[0]) + p_ref[0] + p_ref[1]
    h1 = lax.dot_general(h, w1_ref[...], (((1,), (1,)), ((), ())),
                         preferred_element_type=jnp.float32) + b1_ref[...]
    mean = jnp.mean(h1, axis=0, keepdims=True)
    var = jnp.mean(jnp.square(h1 - mean), axis=0, keepdims=True)
    h2 = (h1 - mean) * lax.rsqrt(var + 1e-5) * g_ref[...] + be_ref[...]
    h2 = jnp.maximum(h2, 0.0)
    o_ref[...] = lax.dot_general(h2, w2_ref[...], (((1,), (1,)), ((), ())),
                                 preferred_element_type=jnp.float32) + b2_ref[...]


def _tc_mlp(eps, x, partials, w1, b1, gamma, beta, w2, b2):
    return pl.pallas_call(
        _tc_mlp_body,
        out_shape=jax.ShapeDtypeStruct((N_NODES, DIM), jnp.float32),
        in_specs=[
            pl.BlockSpec(memory_space=pltpu.SMEM),
            pl.BlockSpec(memory_space=pltpu.VMEM),
            pl.BlockSpec(memory_space=pltpu.VMEM),
            pl.BlockSpec(memory_space=pltpu.VMEM),
            pl.BlockSpec(memory_space=pltpu.VMEM),
            pl.BlockSpec(memory_space=pltpu.VMEM),
            pl.BlockSpec(memory_space=pltpu.VMEM),
            pl.BlockSpec(memory_space=pltpu.VMEM),
            pl.BlockSpec(memory_space=pltpu.VMEM),
        ],
        out_specs=pl.BlockSpec(memory_space=pltpu.VMEM),
    )(eps, x, partials, w1, b1, gamma, beta, w2, b2)


def kernel(x, edge_index, edge_attr, eps, W1, b1, gamma, beta, W2, b2):
    dst = edge_index[0]
    src = edge_index[1]
    partials = _sc_edge(x, src, dst, edge_attr)
    return _tc_mlp(eps, x, partials, W1,
                   b1.reshape(1, DIM), gamma.reshape(1, DIM),
                   beta.reshape(1, DIM), W2, b2.reshape(1, DIM))
